# baseline (speedup 1.0000x reference)
import functools

import jax
import jax.numpy as jnp
from jax import lax
from jax.experimental import pallas as pl
from jax.experimental.pallas import tpu as pltpu
from jax.experimental.pallas import tpu_sc as plsc

B, H, W, C = 64, 128, 128, 4
HW = H * W
BIG = 0x3FFFFFFF
BATCHES_PER_TILE = 2
OPP_CHUNK = 8

_mesh = plsc.VectorSubcoreMesh(core_axis_name="c", subcore_axis_name="s")


@functools.partial(
    pl.kernel,
    out_type=jax.ShapeDtypeStruct((B, H, W), jnp.float32),
    mesh=_mesh,
    scratch_types=[
        pltpu.VMEM((H, W), jnp.float32),
        pltpu.VMEM((H, W), jnp.float32),
        pltpu.VMEM((H, W), jnp.float32),
        pltpu.VMEM((H, W), jnp.float32),
        pltpu.SemaphoreType.DMA,
        pltpu.SemaphoreType.DMA,
        pltpu.SemaphoreType.DMA,
        pltpu.SemaphoreType.DMA,
    ],
    compiler_params=pltpu.CompilerParams(
        needs_layout_passes=False,
        use_tc_tiling_on_sc=True,
    ),
)
def _oracle(p_hbm, g_hbm, pv0, pv1, gb0, gb1, sp0, sp1, sg0, sg1):
    cid = lax.axis_index("c")
    sid = lax.axis_index("s")
    wid = sid * 2 + cid

    lanes = lax.iota(jnp.int32, 16)
    one16 = jnp.full((16,), 1, jnp.int32)
    big16 = jnp.full((16,), BIG, jnp.int32)
    neg16 = jnp.full((16,), -10.0, jnp.float32)
    ten16 = jnp.full((16,), 10.0, jnp.float32)

    pvs, gbs = [pv0, pv1], [gb0, gb1]
    sps, sgs = [sp0, sp1], [sg0, sg1]
    bs = [wid * BATCHES_PER_TILE + bi for bi in range(BATCHES_PER_TILE)]
    cps = [pltpu.async_copy(p_hbm.at[bs[i]], pvs[i], sps[i])
           for i in range(BATCHES_PER_TILE)]
    cgs = []

    for bi in range(BATCHES_PER_TILE):
        b = bs[bi]
        pv, gbuf = pvs[bi], gbs[bi]
        cps[bi].wait()

        def opp_cond(carry):
            i, oppacc = carry
            return (i < HW // 16) & (jnp.min(oppacc) >= BIG)

        def opp_body(carry):
            i, oppacc = carry
            for k in range(OPP_CHUNK):
                step = i + k
                v = pv[step >> 3, pl.ds((step & 7) * 16, 16)].astype(jnp.int32)
                cellidx = step * 16 + lanes
                oppacc = jnp.minimum(
                    oppacc, jnp.where((v >> 4) == one16, cellidx, big16))
            return i + OPP_CHUNK, oppacc

        _, oppacc = lax.while_loop(opp_cond, opp_body, (jnp.int32(0), big16))
        oppidx = jnp.min(oppacc)
        opp_exists = oppidx < BIG
        opp_r = oppidx >> 7
        opp_c = oppidx & 127
        opp_is_start = oppidx == 3 * W + 6

        colkeys = []
        for k in range(8):
            col = 16 * k + lanes
            dc = col - opp_c
            colkeys.append(((dc * dc) << 14) + col)

        def main_row(gr, carry):
            m1, m2 = carry
            dr = gr - opp_r
            rkb = (dr * dr << 14) + gr * 128
            for k in range(8):
                v = pv[gr, pl.ds(16 * k, 16)].astype(jnp.int32)
                eq = (v & 15) == one16
                key = jnp.where(eq, colkeys[k] + rkb, big16)
                m2 = jnp.minimum(m2, jnp.maximum(m1, key))
                m1 = jnp.minimum(m1, key)
            return m1, m2

        m1, m2 = lax.fori_loop(0, H, main_row, (big16, big16), unroll=2)

        m1s = jnp.min(m1)
        m1_excl = jnp.where(m1 == m1s, big16, m1)
        m2s = jnp.minimum(jnp.min(m1_excl), jnp.min(m2))
        d2_min = m1s >> 14
        mi = m1s & 16383
        m_gap = (m2s >> 14) - d2_min
        mg = jnp.minimum(m_gap, 36)
        diff_lt = (m_gap == 0) | (
            (m_gap <= 35) & (10000 * mg * mg - 200 * mg + 1 < 400 * d2_min)
        )
        cond_a = (m2s < BIG) & opp_exists & jnp.logical_not(opp_is_start)
        choose_min = cond_a & jnp.logical_not(diff_lt)

        @pl.when(choose_min)
        def _nearest_only():
            def memset(r, carry):
                for k in range(8):
                    gbuf[r, pl.ds(16 * k, 16)] = neg16
                return carry
            lax.fori_loop(0, H, memset, 0, unroll=4)
            plsc.store_scatter(
                gbuf,
                [jnp.broadcast_to(mi >> 7, (16,)),
                 jnp.broadcast_to(mi & 127, (16,))],
                ten16, mask=lanes == 0)

        @pl.when(jnp.logical_not(choose_min))
        def _food_map():
            def emit(r, carry):
                for k in range(8):
                    v = pv[r, pl.ds(16 * k, 16)].astype(jnp.int32)
                    eq = (v & 15) == one16
                    gbuf[r, pl.ds(16 * k, 16)] = jnp.where(eq, ten16, neg16)
                return carry
            lax.fori_loop(0, H, emit, 0, unroll=2)

        cgs.append(pltpu.async_copy(gbuf, g_hbm.at[b], sgs[bi]))

    for cg in cgs:
        cg.wait()


def kernel(x, history):
    del history
    packed = x[:, :, :, 1] + 16.0 * x[:, :, :, 3]
    return _oracle(packed)
